# baseline (speedup 1.0000x reference)
import functools

import jax
import jax.numpy as jnp
from jax import lax
from jax.experimental import pallas as pl
from jax.experimental.pallas import tpu as pltpu
from jax.experimental.pallas import tpu_sc as plsc

_NC = 2
_NS = 16
_NW = _NC * _NS
_NBUF = 5


@functools.partial(jax.jit, static_argnums=(2, 3, 4))
def _sc_gather(idx_t, table, b_sz, s_sz, d):
  bw = b_sz // _NW
  mesh = plsc.VectorSubcoreMesh(core_axis_name="c", subcore_axis_name="s")

  @functools.partial(
      pl.kernel,
      mesh=mesh,
      out_type=jax.ShapeDtypeStruct((s_sz, b_sz, d), jnp.float32),
      scratch_types=[
          pltpu.VMEM((s_sz, bw), jnp.int32),
          pltpu.VMEM((_NBUF, bw, d), jnp.float32),
      ] + [pltpu.SemaphoreType.DMA] * (2 * _NBUF),
      compiler_params=pltpu.CompilerParams(skip_device_barrier=True),
  )
  def k(idx_hbm, table_hbm, out_hbm, idx_v, rows_v, *sems):
    gsems = sems[:_NBUF]
    psems = sems[_NBUF:]
    wid = lax.axis_index("s") * _NC + lax.axis_index("c")
    base = wid * bw
    pltpu.sync_copy(idx_hbm.at[:, pl.ds(base, bw)], idx_v)

    def start_gather(s, b):
      pltpu.async_copy(table_hbm.at[idx_v.at[s]], rows_v.at[b], gsems[b])

    def wait_gather(s, b):
      pltpu.make_async_copy(
          table_hbm.at[idx_v.at[s]], rows_v.at[b], gsems[b]).wait()

    def start_put(s, b):
      pltpu.async_copy(
          rows_v.at[b], out_hbm.at[s, pl.ds(base, bw)], psems[b])

    def wait_put(s, b):
      pltpu.make_async_copy(
          rows_v.at[b], out_hbm.at[s, pl.ds(base, bw)], psems[b]).wait()

    for b in range(_NBUF):
      start_gather(b, b)

    def body(g, carry):
      for b in range(_NBUF):
        s = g * _NBUF + b
        wait_gather(s, b)
        start_put(s, b)
        if b > 0:
          wait_put(s - 1, b - 1)
          start_gather(s - 1 + _NBUF, b - 1)
      last = g * _NBUF + _NBUF - 1
      wait_put(last, _NBUF - 1)
      start_gather(last + _NBUF, _NBUF - 1)
      return carry

    ngroups = s_sz // _NBUF
    lax.fori_loop(0, ngroups - 1, body, 0)

    for b in range(_NBUF):
      s = (ngroups - 1) * _NBUF + b
      wait_gather(s, b)
      start_put(s, b)
      if b > 0:
        wait_put(s - 1, b - 1)
    wait_put(s_sz - 1, _NBUF - 1)

  return k(idx_t, table)


def kernel(indices, table):
  b_sz, s_sz = indices.shape
  v, d = table.shape
  assert b_sz % _NW == 0 and s_sz % _NBUF == 0, (b_sz, s_sz)
  out_t = _sc_gather(indices.T.astype(jnp.int32), table, b_sz, s_sz, d)
  return jnp.transpose(out_t, (1, 0, 2))
